# Initial kernel scaffold; baseline (speedup 1.0000x reference)
import jax
import jax.numpy as jnp
from jax.experimental import pallas as pl


def kernel(x, params, clusters_0, clusters_1, clusters_2, clusters_3, clusters_4, clusters_5, edge_index_0, edge_index_1, edge_index_2, edge_index_3, edge_index_4, edge_index_5, edge_index_6, selections_0, selections_1, selections_2, selections_3, selections_4, selections_5, selections_6):
    raise NotImplementedError("write your pallas kernel here")


# speedup vs baseline: 1.0050x; 1.0050x over previous
import functools

import jax
import jax.numpy as jnp
from jax.experimental import pallas as pl

NODE_COUNTS = [50000, 25000, 12500, 6250, 3125, 1563, 782]


def _mm_bias_kernel(x_ref, w_ref, b_ref, o_ref):
    o_ref[...] = (
        jnp.dot(x_ref[...], w_ref[...], preferred_element_type=jnp.float32)
        + b_ref[...]
    )


@functools.partial(jax.jit, static_argnames=())
def _mm_bias(x, w, b):
    n, k = x.shape
    o = w.shape[1]
    bn = 256
    n_pad = ((n + bn - 1) // bn) * bn
    xp = jnp.pad(x, ((0, n_pad - n), (0, 0)))
    b2 = b.reshape(1, o)
    out = pl.pallas_call(
        _mm_bias_kernel,
        grid=(n_pad // bn,),
        in_specs=[
            pl.BlockSpec((bn, k), lambda i: (i, 0)),
            pl.BlockSpec((k, o), lambda i: (0, 0)),
            pl.BlockSpec((1, o), lambda i: (0, 0)),
        ],
        out_specs=pl.BlockSpec((bn, o), lambda i: (i, 0)),
        out_shape=jax.ShapeDtypeStruct((n_pad, o), jnp.float32),
    )(xp, w, b2)
    return out[:n]


def _sel_conv(p, x, ei, sel):
    w = p["w"]
    b = p["b"]
    s = w.shape[0]
    n = x.shape[0]
    cin = x.shape[1]
    if s == 1:
        return _mm_bias(x, w[0], b)
    src = ei[0]
    dst = ei[1]
    msg = x[src]
    seg = dst * s + sel
    agg = jax.ops.segment_sum(msg, seg, num_segments=n * s)
    return _mm_bias(agg.reshape(n, s * cin), w.reshape(s * cin, -1), b)


def _batchnorm(p, x, eps=1e-5):
    m = jnp.mean(x, axis=0)
    v = jnp.var(x, axis=0)
    return (x - m) / jnp.sqrt(v + eps) * p["g"] + p["bt"]


def _stride_pool(x, cluster, nc):
    sums = jax.ops.segment_sum(x, cluster, num_segments=nc)
    cnt = jax.ops.segment_sum(jnp.ones((x.shape[0], 1), x.dtype), cluster, num_segments=nc)
    return sums / jnp.maximum(cnt, 1.0)


def _conv_fwd(p, x, ei, sel, cluster=None, nc=None):
    h = _sel_conv(p["sc"], x, ei, sel)
    if cluster is not None:
        h = _stride_pool(h, cluster, nc)
    return jax.nn.elu(_batchnorm(p["bn"], h))


def _resconv_fwd(p, x, ei, sel, cluster=None, nc=None, dei=None, dsel=None):
    h = _conv_fwd(p["c1"], x, ei, sel)
    if cluster is not None:
        h = _stride_pool(h, cluster, nc)
        h = _conv_fwd(p["c2"], h, dei, dsel)
        sc = _stride_pool(_sel_conv(p["c3"], x, ei, sel), cluster, nc)
    else:
        h = _conv_fwd(p["c2"], h, ei, sel)
        sc = _sel_conv(p["c3"], x, ei, sel)
    return jax.nn.elu(_batchnorm(p["bn"], h + sc))


def _max_pool(x, ei, sel, cluster, nc):
    src = ei[0]
    dst = ei[1]
    nm = jax.ops.segment_max(x[src], dst, num_segments=x.shape[0])
    nm = jnp.maximum(nm, x)
    pooled = jax.ops.segment_max(nm, cluster, num_segments=nc)
    return jnp.where(jnp.isfinite(pooled), pooled, 0.0)


def _unpool_bilinear(x, cluster, ei, sel):
    xf = x[cluster]
    src = ei[0]
    dst = ei[1]
    nf = xf.shape[0]
    s = jax.ops.segment_sum(xf[src], dst, num_segments=nf)
    c = jax.ops.segment_sum(jnp.ones((ei.shape[1], 1), x.dtype), dst, num_segments=nf)
    mean = s / jnp.maximum(c, 1.0)
    return jnp.where(c > 0, 0.5 * xf + 0.5 * mean, xf)


def _upconv_fwd(p, x, cluster, ei, sel):
    return _conv_fwd(p, _unpool_bilinear(x, cluster, ei, sel), ei, sel)


def _get_disp_fwd(p, x, ei, sel):
    return 0.3 * jax.nn.sigmoid(_batchnorm(p["bn"], _sel_conv(p["sc"], x, ei, sel)))


def kernel(x, params, clusters_0, clusters_1, clusters_2, clusters_3, clusters_4, clusters_5, edge_index_0, edge_index_1, edge_index_2, edge_index_3, edge_index_4, edge_index_5, edge_index_6, selections_0, selections_1, selections_2, selections_3, selections_4, selections_5, selections_6):
    clusters = [clusters_0, clusters_1, clusters_2, clusters_3, clusters_4, clusters_5]
    eis = [edge_index_0, edge_index_1, edge_index_2, edge_index_3, edge_index_4, edge_index_5, edge_index_6]
    sels = [selections_0, selections_1, selections_2, selections_3, selections_4, selections_5, selections_6]
    nc = NODE_COUNTS

    x1 = _conv_fwd(params["conv1"], x, eis[0], sels[0], clusters[0], nc[1])
    xp1 = _max_pool(x1, eis[1], sels[1], clusters[1], nc[2])
    x2 = _resconv_fwd(params["down2"], xp1, eis[2], sels[2], clusters[2], nc[3], eis[3], sels[3])
    x2 = _resconv_fwd(params["conv2"], x2, eis[3], sels[3])
    x3 = _resconv_fwd(params["down3"], x2, eis[3], sels[3], clusters[3], nc[4], eis[4], sels[4])
    x3 = _resconv_fwd(params["conv3"], x3, eis[4], sels[4])
    x4 = _resconv_fwd(params["down4"], x3, eis[4], sels[4], clusters[4], nc[5], eis[5], sels[5])
    x4 = _resconv_fwd(params["conv4"], x4, eis[5], sels[5])
    x5 = _resconv_fwd(params["down5"], x4, eis[5], sels[5], clusters[5], nc[6], eis[6], sels[6])
    x5 = _resconv_fwd(params["conv5"], x5, eis[6], sels[6])
    up6 = _upconv_fwd(params["upconv6"], x5, clusters[5], eis[5], sels[5])
    i6 = _conv_fwd(params["iconv6"], jnp.concatenate([up6, x4], 1), eis[5], sels[5])
    up5 = _upconv_fwd(params["upconv5"], i6, clusters[4], eis[4], sels[4])
    i5 = _conv_fwd(params["iconv5"], jnp.concatenate([up5, x3], 1), eis[4], sels[4])
    up4 = _upconv_fwd(params["upconv4"], i5, clusters[3], eis[3], sels[3])
    i4 = _conv_fwd(params["iconv4"], jnp.concatenate([up4, x2], 1), eis[3], sels[3])
    disp4 = _get_disp_fwd(params["disp4"], i4, eis[3], sels[3])
    ud4 = _unpool_bilinear(disp4, clusters[2], eis[2], sels[2])
    up3 = _upconv_fwd(params["upconv3"], i4, clusters[2], eis[2], sels[2])
    i3 = _conv_fwd(params["iconv3"], jnp.concatenate([up3, xp1, ud4], 1), eis[2], sels[2])
    disp3 = _get_disp_fwd(params["disp3"], i3, eis[2], sels[2])
    ud3 = _unpool_bilinear(disp3, clusters[1], eis[1], sels[1])
    up2 = _upconv_fwd(params["upconv2"], i3, clusters[1], eis[1], sels[1])
    i2 = _conv_fwd(params["iconv2"], jnp.concatenate([up2, x1, ud3], 1), eis[1], sels[1])
    disp2 = _get_disp_fwd(params["disp2"], i2, eis[1], sels[1])
    ud2 = _unpool_bilinear(disp2, clusters[0], eis[0], sels[0])
    up1 = _upconv_fwd(params["upconv1"], i2, clusters[0], eis[0], sels[0])
    i1 = _conv_fwd(params["iconv1"], jnp.concatenate([up1, ud2], 1), eis[0], sels[0])
    disp1 = _get_disp_fwd(params["disp1"], i1, eis[0], sels[0])
    return (disp1, disp2, disp3, disp4)


# speedup vs baseline: 1.1706x; 1.1647x over previous
import functools

import jax
import jax.numpy as jnp
from jax.experimental import pallas as pl

NODE_COUNTS = [50000, 25000, 12500, 6250, 3125, 1563, 782]


def _mm_bias_kernel(x_ref, w_ref, b_ref, o_ref):
    o_ref[...] = (
        jnp.dot(x_ref[...], w_ref[...], preferred_element_type=jnp.float32)
        + b_ref[...]
    )


@functools.partial(jax.jit, static_argnames=())
def _mm_bias(x, w, b):
    n, k = x.shape
    o = w.shape[1]
    bn = 256
    n_pad = ((n + bn - 1) // bn) * bn
    xp = jnp.pad(x, ((0, n_pad - n), (0, 0)))
    b2 = b.reshape(1, o)
    out = pl.pallas_call(
        _mm_bias_kernel,
        grid=(n_pad // bn,),
        in_specs=[
            pl.BlockSpec((bn, k), lambda i: (i, 0)),
            pl.BlockSpec((k, o), lambda i: (0, 0)),
            pl.BlockSpec((1, o), lambda i: (0, 0)),
        ],
        out_specs=pl.BlockSpec((bn, o), lambda i: (i, 0)),
        out_shape=jax.ShapeDtypeStruct((n_pad, o), jnp.float32),
    )(xp, w, b2)
    return out[:n]


def _sel_conv(p, x, ei, sel):
    w = p["w"]
    b = p["b"]
    s = w.shape[0]
    n = x.shape[0]
    cin = x.shape[1]
    cout = w.shape[2]
    if s == 1:
        return _mm_bias(x, w[0], b)
    src = ei[0]
    dst = ei[1]
    if 26 * cout < 34 * cin:
        w_all = w.transpose(1, 0, 2).reshape(cin, s * cout)
        xw = _mm_bias(x, w_all, jnp.zeros((s * cout,), jnp.float32))
        xw_flat = xw.reshape(n * s, cout)
        msg = xw_flat[src * s + sel]
        return jax.ops.segment_sum(msg, dst, num_segments=n) + b
    msg = x[src]
    seg = dst * s + sel
    agg = jax.ops.segment_sum(msg, seg, num_segments=n * s)
    return _mm_bias(agg.reshape(n, s * cin), w.reshape(s * cin, -1), b)


def _batchnorm(p, x, eps=1e-5):
    m = jnp.mean(x, axis=0)
    v = jnp.var(x, axis=0)
    return (x - m) / jnp.sqrt(v + eps) * p["g"] + p["bt"]


def _stride_pool(x, cluster, nc):
    sums = jax.ops.segment_sum(x, cluster, num_segments=nc)
    cnt = jax.ops.segment_sum(jnp.ones((x.shape[0], 1), x.dtype), cluster, num_segments=nc)
    return sums / jnp.maximum(cnt, 1.0)


def _conv_fwd(p, x, ei, sel, cluster=None, nc=None):
    h = _sel_conv(p["sc"], x, ei, sel)
    if cluster is not None:
        h = _stride_pool(h, cluster, nc)
    return jax.nn.elu(_batchnorm(p["bn"], h))


def _resconv_fwd(p, x, ei, sel, cluster=None, nc=None, dei=None, dsel=None):
    h = _conv_fwd(p["c1"], x, ei, sel)
    if cluster is not None:
        h = _stride_pool(h, cluster, nc)
        h = _conv_fwd(p["c2"], h, dei, dsel)
        sc = _stride_pool(_sel_conv(p["c3"], x, ei, sel), cluster, nc)
    else:
        h = _conv_fwd(p["c2"], h, ei, sel)
        sc = _sel_conv(p["c3"], x, ei, sel)
    return jax.nn.elu(_batchnorm(p["bn"], h + sc))


def _max_pool(x, ei, sel, cluster, nc):
    src = ei[0]
    dst = ei[1]
    nm = jax.ops.segment_max(x[src], dst, num_segments=x.shape[0])
    nm = jnp.maximum(nm, x)
    pooled = jax.ops.segment_max(nm, cluster, num_segments=nc)
    return jnp.where(jnp.isfinite(pooled), pooled, 0.0)


def _unpool_bilinear(x, cluster, ei, sel):
    xf = x[cluster]
    src = ei[0]
    dst = ei[1]
    nf = xf.shape[0]
    s = jax.ops.segment_sum(xf[src], dst, num_segments=nf)
    c = jax.ops.segment_sum(jnp.ones((ei.shape[1], 1), x.dtype), dst, num_segments=nf)
    mean = s / jnp.maximum(c, 1.0)
    return jnp.where(c > 0, 0.5 * xf + 0.5 * mean, xf)


def _upconv_fwd(p, x, cluster, ei, sel):
    return _conv_fwd(p, _unpool_bilinear(x, cluster, ei, sel), ei, sel)


def _get_disp_fwd(p, x, ei, sel):
    return 0.3 * jax.nn.sigmoid(_batchnorm(p["bn"], _sel_conv(p["sc"], x, ei, sel)))


def kernel(x, params, clusters_0, clusters_1, clusters_2, clusters_3, clusters_4, clusters_5, edge_index_0, edge_index_1, edge_index_2, edge_index_3, edge_index_4, edge_index_5, edge_index_6, selections_0, selections_1, selections_2, selections_3, selections_4, selections_5, selections_6):
    clusters = [clusters_0, clusters_1, clusters_2, clusters_3, clusters_4, clusters_5]
    eis = [edge_index_0, edge_index_1, edge_index_2, edge_index_3, edge_index_4, edge_index_5, edge_index_6]
    sels = [selections_0, selections_1, selections_2, selections_3, selections_4, selections_5, selections_6]
    nc = NODE_COUNTS

    x1 = _conv_fwd(params["conv1"], x, eis[0], sels[0], clusters[0], nc[1])
    xp1 = _max_pool(x1, eis[1], sels[1], clusters[1], nc[2])
    x2 = _resconv_fwd(params["down2"], xp1, eis[2], sels[2], clusters[2], nc[3], eis[3], sels[3])
    x2 = _resconv_fwd(params["conv2"], x2, eis[3], sels[3])
    x3 = _resconv_fwd(params["down3"], x2, eis[3], sels[3], clusters[3], nc[4], eis[4], sels[4])
    x3 = _resconv_fwd(params["conv3"], x3, eis[4], sels[4])
    x4 = _resconv_fwd(params["down4"], x3, eis[4], sels[4], clusters[4], nc[5], eis[5], sels[5])
    x4 = _resconv_fwd(params["conv4"], x4, eis[5], sels[5])
    x5 = _resconv_fwd(params["down5"], x4, eis[5], sels[5], clusters[5], nc[6], eis[6], sels[6])
    x5 = _resconv_fwd(params["conv5"], x5, eis[6], sels[6])
    up6 = _upconv_fwd(params["upconv6"], x5, clusters[5], eis[5], sels[5])
    i6 = _conv_fwd(params["iconv6"], jnp.concatenate([up6, x4], 1), eis[5], sels[5])
    up5 = _upconv_fwd(params["upconv5"], i6, clusters[4], eis[4], sels[4])
    i5 = _conv_fwd(params["iconv5"], jnp.concatenate([up5, x3], 1), eis[4], sels[4])
    up4 = _upconv_fwd(params["upconv4"], i5, clusters[3], eis[3], sels[3])
    i4 = _conv_fwd(params["iconv4"], jnp.concatenate([up4, x2], 1), eis[3], sels[3])
    disp4 = _get_disp_fwd(params["disp4"], i4, eis[3], sels[3])
    ud4 = _unpool_bilinear(disp4, clusters[2], eis[2], sels[2])
    up3 = _upconv_fwd(params["upconv3"], i4, clusters[2], eis[2], sels[2])
    i3 = _conv_fwd(params["iconv3"], jnp.concatenate([up3, xp1, ud4], 1), eis[2], sels[2])
    disp3 = _get_disp_fwd(params["disp3"], i3, eis[2], sels[2])
    ud3 = _unpool_bilinear(disp3, clusters[1], eis[1], sels[1])
    up2 = _upconv_fwd(params["upconv2"], i3, clusters[1], eis[1], sels[1])
    i2 = _conv_fwd(params["iconv2"], jnp.concatenate([up2, x1, ud3], 1), eis[1], sels[1])
    disp2 = _get_disp_fwd(params["disp2"], i2, eis[1], sels[1])
    ud2 = _unpool_bilinear(disp2, clusters[0], eis[0], sels[0])
    up1 = _upconv_fwd(params["upconv1"], i2, clusters[0], eis[0], sels[0])
    i1 = _conv_fwd(params["iconv1"], jnp.concatenate([up1, ud2], 1), eis[0], sels[0])
    disp1 = _get_disp_fwd(params["disp1"], i1, eis[0], sels[0])
    return (disp1, disp2, disp3, disp4)
